# baseline (speedup 1.0000x reference)
import functools

import jax
import jax.numpy as jnp
from jax import lax
from jax.experimental import pallas as pl
from jax.experimental.pallas import tpu as pltpu
from jax.experimental.pallas import tpu_sc as plsc

VOCAB = 1000000
HIDDEN = 64
BATCH = 16384
FIELDS = 26

_NW = 32
_NBLK = 7813
_VPAD = _NBLK * 128
_FROW = _VPAD // 2
_MAGIC = 123456.75
_BW = BATCH // _NW
_CH = 128
_NSUB = _BW // _CH
_NSTEP = FIELDS * _NSUB
_PER_W = FIELDS * _BW


@functools.partial(
    pl.kernel,
    mesh=plsc.VectorSubcoreMesh(core_axis_name="c", subcore_axis_name="s"),
    out_type=(
        jax.ShapeDtypeStruct((FIELDS, HIDDEN, BATCH), jnp.float32),
        jax.ShapeDtypeStruct((_VPAD // 2 + _NW, 2 * HIDDEN), jnp.float32),
    ),
    scratch_types=[
        pltpu.VMEM((2, HIDDEN, _CH), jnp.float32),
        pltpu.VMEM((2, _CH // 2, 2 * HIDDEN), jnp.float32),
        pltpu.VMEM((_PER_W,), jnp.int32),
        pltpu.VMEM((_PER_W,), jnp.int32),
        pltpu.VMEM((2, _CH, 2 * HIDDEN), jnp.float32),
        pltpu.VMEM((2, HIDDEN, _CH), jnp.float32),
        pltpu.VMEM((2 * HIDDEN,), jnp.float32),
        pltpu.VMEM((_NW, 2 * HIDDEN), jnp.float32),
        pltpu.SemaphoreType.DMA,
        pltpu.SemaphoreType.DMA((2,)),
        pltpu.SemaphoreType.DMA((2,)),
        pltpu.SemaphoreType.DMA((2,)),
        pltpu.SemaphoreType.DMA((2,)),
    ],
    compiler_params=pltpu.CompilerParams(needs_layout_passes=False),
)
def _emb_lookup(xlin_hbm, tablet_hbm, out_hbm, scratch_hbm,
                tin_v, tout_v, idx_all, p_all, rows_v, cols_v, flag_v, poll_v,
                i_sem, ti_sem, to_sem, g_sem, o_sem):
    wid = lax.axis_index("s") * 2 + lax.axis_index("c")
    base_b = wid * _BW
    iot = lax.iota(jnp.int32, 16)

    idx_descs = []
    for f in range(FIELDS):
        idx_descs.append(pltpu.async_copy(
            xlin_hbm.at[pl.ds(f * BATCH + base_b, _BW)],
            idx_all.at[pl.ds(f * _BW, _BW)],
            i_sem,
        ))

    nb = jnp.where(wid < _NBLK - 244 * _NW, 245, 244)

    def t_in(k, b):
        bid = wid + k * _NW
        for th in range(HIDDEN // 8):
            pltpu.async_copy(
                tablet_hbm.at[pl.ds(th * 8, 8), pl.ds(bid * _CH, _CH)],
                tin_v.at[b, pl.ds(th * 8, 8)],
                ti_sem.at[b],
            )

    def t_in_wait(k, b):
        bid = wid + k * _NW
        for th in range(HIDDEN // 8):
            pltpu.make_async_copy(
                tablet_hbm.at[pl.ds(th * 8, 8), pl.ds(bid * _CH, _CH)],
                tin_v.at[b, pl.ds(th * 8, 8)],
                ti_sem.at[b],
            ).wait()

    def t_out(k, b):
        bid = wid + k * _NW
        return pltpu.async_copy(
            tout_v.at[b],
            scratch_hbm.at[pl.ds(bid * (_CH // 2), _CH // 2), :],
            to_sem.at[b],
        )

    def t_out_wait(k, b):
        bid = wid + k * _NW
        pltpu.make_async_copy(
            tout_v.at[b],
            scratch_hbm.at[pl.ds(bid * (_CH // 2), _CH // 2), :],
            to_sem.at[b],
        ).wait()

    t_in(0, 0)

    def t_body(k, carry):
        b = lax.rem(k, 2)

        @pl.when(k + 1 < nb)
        def _():
            t_in(k + 1, 1 - b)

        @pl.when(k < nb)
        def _():
            t_in_wait(k, b)

            @pl.when(k >= 2)
            def _():
                t_out_wait(k - 2, b)

            @plsc.parallel_loop(0, _CH // 2, unroll=2)
            def _(p):
                for q in range(8):
                    hvec = iot + (q % 4) * 16
                    cfull = jnp.full((16,), 0, jnp.int32) + (2 * p + q // 4)
                    vals = plsc.load_gather(tin_v.at[b], [hvec, cfull])
                    tout_v[b, p, pl.ds(q * 16, 16)] = vals

            t_out(k, b)
        return carry

    lax.fori_loop(0, 245, t_body, 0)
    t_out_wait(nb - 2, lax.rem(nb - 2, 2))
    t_out_wait(nb - 1, lax.rem(nb - 1, 2))

    for q in range((2 * HIDDEN) // 16):
        flag_v[pl.ds(q * 16, 16)] = jnp.full((16,), _MAGIC, jnp.float32)
    pltpu.sync_copy(flag_v, scratch_hbm.at[_FROW + wid, :])

    def poll_body(carry):
        pltpu.sync_copy(scratch_hbm.at[pl.ds(_FROW, _NW), :], poll_v)
        acc = jnp.zeros((16,), jnp.int32)
        for r in range(_NW):
            acc = acc | (poll_v[r, pl.ds(0, 16)] != _MAGIC).astype(jnp.int32)
        return jnp.max(acc) != 0

    lax.while_loop(lambda c: c, poll_body, True)

    for d in idx_descs:
        d.wait()

    def prep(i, carry):
        v = idx_all[pl.ds(i * 16, 16)]
        p_all[pl.ds(i * 16, 16)] = lax.shift_right_logical(v, 1)
        return carry

    lax.fori_loop(0, _PER_W // 16, prep, 0)

    def gstart(s, b):
        return pltpu.async_copy(
            scratch_hbm.at[p_all.at[pl.ds(s * _CH, _CH)]],
            rows_v.at[b],
            g_sem.at[b],
        )

    def gwait(s, b):
        pltpu.make_async_copy(
            scratch_hbm.at[p_all.at[pl.ds(s * _CH, _CH)]],
            rows_v.at[b],
            g_sem.at[b],
        ).wait()

    def owait(s, ob):
        f = s // _NSUB
        sub = lax.rem(s, _NSUB)
        pltpu.make_async_copy(
            cols_v.at[ob],
            out_hbm.at[f, :, pl.ds(base_b + sub * _CH, _CH)],
            o_sem.at[ob],
        ).wait()

    def shuffle_and_write(s, b, ob):
        @plsc.parallel_loop(0, _CH // 16, unroll=2)
        def _(cb):
            cvec = iot + cb * 16
            colbase = (idx_all[pl.ds(s * _CH + cb * 16, 16)] & 1) * HIDDEN
            for h in range(HIDDEN):
                vals = plsc.load_gather(rows_v.at[b], [cvec, colbase + h])
                cols_v[ob, h, pl.ds(cb * 16, 16)] = vals

        f = s // _NSUB
        sub = lax.rem(s, _NSUB)
        pltpu.async_copy(
            cols_v.at[ob],
            out_hbm.at[f, :, pl.ds(base_b + sub * _CH, _CH)],
            o_sem.at[ob],
        )

    gstart(0, 0)
    gstart(1, 1)

    def body(s, carry):
        b = lax.rem(s, 2)
        ob = lax.rem(s, 2)

        gwait(s, b)

        @pl.when(s >= 2)
        def _():
            owait(s - 2, ob)

        shuffle_and_write(s, b, ob)

        @pl.when(s + 2 < _NSTEP)
        def _():
            gstart(s + 2, b)
        return carry

    lax.fori_loop(0, _NSTEP, body, 0)
    owait(_NSTEP - 2, 0)
    owait(_NSTEP - 1, 1)

    for q in range((2 * HIDDEN) // 16):
        flag_v[pl.ds(q * 16, 16)] = jnp.zeros((16,), jnp.float32)
    pltpu.sync_copy(flag_v, scratch_hbm.at[_FROW + wid, :])


def kernel(x, embedding):
    xlin = x.T.reshape(-1)
    tablet = embedding.T
    out_phys, _ = _emb_lookup(xlin, tablet)
    return out_phys.transpose(2, 0, 1)


# speedup vs baseline: 1.0056x; 1.0056x over previous
import functools

import jax
import jax.numpy as jnp
from jax import lax
from jax.experimental import pallas as pl
from jax.experimental.pallas import tpu as pltpu
from jax.experimental.pallas import tpu_sc as plsc

VOCAB = 1000000
HIDDEN = 64
BATCH = 16384
FIELDS = 26

_NW = 32
_NBLK = 7813
_VPAD = _NBLK * 128
_FROW = _VPAD // 2
_MAGIC = 123456.75
_BW = BATCH // _NW
_CH = 128
_NSUB = _BW // _CH
_NSTEP = FIELDS * _NSUB
_PER_W = FIELDS * _BW


@functools.partial(
    pl.kernel,
    mesh=plsc.VectorSubcoreMesh(core_axis_name="c", subcore_axis_name="s"),
    out_type=(
        jax.ShapeDtypeStruct((FIELDS, HIDDEN, BATCH), jnp.float32),
        jax.ShapeDtypeStruct((_VPAD // 2 + _NW, 2 * HIDDEN), jnp.float32),
    ),
    scratch_types=[
        pltpu.VMEM((2, HIDDEN, _CH), jnp.float32),
        pltpu.VMEM((2, _CH // 2, 2 * HIDDEN), jnp.float32),
        pltpu.VMEM((_PER_W,), jnp.int32),
        pltpu.VMEM((_PER_W,), jnp.int32),
        pltpu.VMEM((2, _CH, 2 * HIDDEN), jnp.float32),
        pltpu.VMEM((2, HIDDEN, _CH), jnp.float32),
        pltpu.VMEM((2 * HIDDEN,), jnp.float32),
        pltpu.VMEM((_NW, 2 * HIDDEN), jnp.float32),
        pltpu.SemaphoreType.DMA,
        pltpu.SemaphoreType.DMA((2,)),
        pltpu.SemaphoreType.DMA((2,)),
        pltpu.SemaphoreType.DMA((2,)),
        pltpu.SemaphoreType.DMA((2,)),
    ],
    compiler_params=pltpu.CompilerParams(needs_layout_passes=False),
)
def _emb_lookup(xlin_hbm, tablet_hbm, out_hbm, scratch_hbm,
                tin_v, tout_v, idx_all, p_all, rows_v, cols_v, flag_v, poll_v,
                i_sem, ti_sem, to_sem, g_sem, o_sem):
    wid = lax.axis_index("s") * 2 + lax.axis_index("c")
    base_b = wid * _BW
    iot = lax.iota(jnp.int32, 16)

    idx_descs = []
    for f in range(FIELDS):
        idx_descs.append(pltpu.async_copy(
            xlin_hbm.at[pl.ds(f * BATCH + base_b, _BW)],
            idx_all.at[pl.ds(f * _BW, _BW)],
            i_sem,
        ))

    nb = jnp.where(wid < _NBLK - 244 * _NW, 245, 244)

    def t_in(k, b):
        bid = wid + k * _NW
        pltpu.async_copy(
            tablet_hbm.at[:, pl.ds(bid * _CH, _CH)],
            tin_v.at[b],
            ti_sem.at[b],
        )

    def t_in_wait(k, b):
        bid = wid + k * _NW
        pltpu.make_async_copy(
            tablet_hbm.at[:, pl.ds(bid * _CH, _CH)],
            tin_v.at[b],
            ti_sem.at[b],
        ).wait()

    def t_out(k, b):
        bid = wid + k * _NW
        return pltpu.async_copy(
            tout_v.at[b],
            scratch_hbm.at[pl.ds(bid * (_CH // 2), _CH // 2), :],
            to_sem.at[b],
        )

    def t_out_wait(k, b):
        bid = wid + k * _NW
        pltpu.make_async_copy(
            tout_v.at[b],
            scratch_hbm.at[pl.ds(bid * (_CH // 2), _CH // 2), :],
            to_sem.at[b],
        ).wait()

    t_in(0, 0)

    def t_body(k, carry):
        b = lax.rem(k, 2)

        @pl.when(k + 1 < nb)
        def _():
            t_in(k + 1, 1 - b)

        @pl.when(k < nb)
        def _():
            t_in_wait(k, b)

            @pl.when(k >= 2)
            def _():
                t_out_wait(k - 2, b)

            @plsc.parallel_loop(0, _CH // 2, unroll=2)
            def _(p):
                for q in range(8):
                    hvec = iot + (q % 4) * 16
                    cfull = jnp.full((16,), 0, jnp.int32) + (2 * p + q // 4)
                    vals = plsc.load_gather(tin_v.at[b], [hvec, cfull])
                    tout_v[b, p, pl.ds(q * 16, 16)] = vals

            t_out(k, b)
        return carry

    lax.fori_loop(0, 245, t_body, 0)
    t_out_wait(nb - 2, lax.rem(nb - 2, 2))
    t_out_wait(nb - 1, lax.rem(nb - 1, 2))

    for q in range((2 * HIDDEN) // 16):
        flag_v[pl.ds(q * 16, 16)] = jnp.full((16,), _MAGIC, jnp.float32)
    pltpu.sync_copy(flag_v, scratch_hbm.at[_FROW + wid, :])

    def poll_body(carry):
        pltpu.sync_copy(scratch_hbm.at[pl.ds(_FROW, _NW), :], poll_v)
        acc = jnp.zeros((16,), jnp.int32)
        for r in range(_NW):
            acc = acc | (poll_v[r, pl.ds(0, 16)] != _MAGIC).astype(jnp.int32)
        return jnp.max(acc) != 0

    lax.while_loop(lambda c: c, poll_body, True)

    for d in idx_descs:
        d.wait()

    def prep(i, carry):
        v = idx_all[pl.ds(i * 16, 16)]
        p_all[pl.ds(i * 16, 16)] = lax.shift_right_logical(v, 1)
        return carry

    lax.fori_loop(0, _PER_W // 16, prep, 0)

    def gstart(s, b):
        return pltpu.async_copy(
            scratch_hbm.at[p_all.at[pl.ds(s * _CH, _CH)]],
            rows_v.at[b],
            g_sem.at[b],
        )

    def gwait(s, b):
        pltpu.make_async_copy(
            scratch_hbm.at[p_all.at[pl.ds(s * _CH, _CH)]],
            rows_v.at[b],
            g_sem.at[b],
        ).wait()

    def owait(s, ob):
        f = s // _NSUB
        sub = lax.rem(s, _NSUB)
        pltpu.make_async_copy(
            cols_v.at[ob],
            out_hbm.at[f, :, pl.ds(base_b + sub * _CH, _CH)],
            o_sem.at[ob],
        ).wait()

    def shuffle_and_write(s, b, ob):
        @plsc.parallel_loop(0, _CH // 16, unroll=2)
        def _(cb):
            cvec = iot + cb * 16
            colbase = (idx_all[pl.ds(s * _CH + cb * 16, 16)] & 1) * HIDDEN
            for h in range(HIDDEN):
                vals = plsc.load_gather(rows_v.at[b], [cvec, colbase + h])
                cols_v[ob, h, pl.ds(cb * 16, 16)] = vals

        f = s // _NSUB
        sub = lax.rem(s, _NSUB)
        pltpu.async_copy(
            cols_v.at[ob],
            out_hbm.at[f, :, pl.ds(base_b + sub * _CH, _CH)],
            o_sem.at[ob],
        )

    gstart(0, 0)
    gstart(1, 1)

    def body(s, carry):
        b = lax.rem(s, 2)
        ob = lax.rem(s, 2)

        gwait(s, b)

        @pl.when(s >= 2)
        def _():
            owait(s - 2, ob)

        shuffle_and_write(s, b, ob)

        @pl.when(s + 2 < _NSTEP)
        def _():
            gstart(s + 2, b)
        return carry

    lax.fori_loop(0, _NSTEP, body, 0)
    owait(_NSTEP - 2, 0)
    owait(_NSTEP - 1, 1)

    for q in range((2 * HIDDEN) // 16):
        flag_v[pl.ds(q * 16, 16)] = jnp.zeros((16,), jnp.float32)
    pltpu.sync_copy(flag_v, scratch_hbm.at[_FROW + wid, :])


def kernel(x, embedding):
    xlin = x.T.reshape(-1)
    tablet = embedding.T
    out_phys, _ = _emb_lookup(xlin, tablet)
    return out_phys.transpose(2, 0, 1)


# speedup vs baseline: 1.2982x; 1.2910x over previous
import functools

import jax
import jax.numpy as jnp
from jax import lax
from jax.experimental import pallas as pl
from jax.experimental.pallas import tpu as pltpu
from jax.experimental.pallas import tpu_sc as plsc

VOCAB = 1000000
HIDDEN = 64
BATCH = 16384
FIELDS = 26

_B = BATCH * FIELDS
_NW = 32
_B_PER_W = _B // _NW
_CHUNK = 832
_NCHUNK = _B_PER_W // _CHUNK
_NBUF = 2


@functools.partial(
    pl.kernel,
    mesh=plsc.VectorSubcoreMesh(core_axis_name="c", subcore_axis_name="s"),
    out_type=jax.ShapeDtypeStruct((_B, HIDDEN), jnp.float32),
    scratch_types=[
        pltpu.VMEM((_B_PER_W,), jnp.int32),
        pltpu.VMEM((_NBUF, _CHUNK, HIDDEN), jnp.float32),
        pltpu.SemaphoreType.DMA((_NBUF,)),
        pltpu.SemaphoreType.DMA((_NBUF,)),
    ],
    compiler_params=pltpu.CompilerParams(use_tc_tiling_on_sc=False),
)
def _emb_lookup(idx_hbm, table_hbm, out_hbm, idx_v, rows_v, gsem, osem):
    wid = lax.axis_index("s") * 2 + lax.axis_index("c")
    base = wid * _B_PER_W

    pltpu.sync_copy(idx_hbm.at[pl.ds(base, _B_PER_W)], idx_v)

    gd = [None] * _NCHUNK
    od = [None] * _NCHUNK
    for g in range(_NCHUNK + 1):
        b = g % _NBUF
        if g < _NCHUNK:
            if g >= _NBUF:
                od[g - _NBUF].wait()
            gd[g] = pltpu.async_copy(
                table_hbm.at[idx_v.at[pl.ds(g * _CHUNK, _CHUNK)]],
                rows_v.at[b],
                gsem.at[b],
            )
        if g >= 1:
            p = g - 1
            gd[p].wait()
            od[p] = pltpu.async_copy(
                rows_v.at[p % _NBUF],
                out_hbm.at[pl.ds(base + p * _CHUNK, _CHUNK)],
                osem.at[p % _NBUF],
            )
    od[_NCHUNK - 2].wait()
    od[_NCHUNK - 1].wait()


def kernel(x, embedding):
    flat = x.reshape(_B)
    out = _emb_lookup(flat, embedding)
    return out.reshape(BATCH, FIELDS, HIDDEN)
